# baseline (speedup 1.0000x reference)
import functools

import jax
import jax.numpy as jnp
from jax import lax
from jax.experimental import pallas as pl
from jax.experimental.pallas import tpu as pltpu
from jax.experimental.pallas import tpu_sc as plsc

_B = 64
_V = 100000
_DEPTH = 4
_TOPP = 0.9

_W = 2000
_NCH = _W // 16
_NBKT = 65536
_NCORE = 2
_NSUB = 16
_NWORK = _NCORE * _NSUB


def _expg_const():
    skey = jax.random.key(12345)
    skeys = jax.random.split(skey, _B)
    g = jax.vmap(lambda k: jax.random.gumbel(k, (_V,), jnp.float32))(skeys)
    return jnp.exp(g)


def _build_sc(interpret=False):
    nwin = _V // _W
    mesh = plsc.VectorSubcoreMesh(
        core_axis_name="c", subcore_axis_name="s",
        num_cores=_NCORE, num_subcores=_NSUB)

    @functools.partial(
        pl.kernel,
        mesh=mesh,
        compiler_params=pltpu.CompilerParams(needs_layout_passes=False),
        out_type=[
            jax.ShapeDtypeStruct((_B * _V,), jnp.float32),
            jax.ShapeDtypeStruct((_B * _V,), jnp.float32),
            jax.ShapeDtypeStruct((_B * _V,), jnp.int32),
            jax.ShapeDtypeStruct((_B * _V,), jnp.float32),
            jax.ShapeDtypeStruct((_B * _V,), jnp.int32),
        ],
        scratch_types=[
            pltpu.VMEM((_NBKT,), jnp.int32),
            pltpu.VMEM((_W,), jnp.float32),
            pltpu.VMEM((_W,), jnp.int32),
            pltpu.VMEM((_W,), jnp.float32),
            pltpu.VMEM((_W,), jnp.float32),
            pltpu.VMEM((_W,), jnp.int32),
            pltpu.VMEM((_W,), jnp.int32),
            pltpu.VMEM((16,), jnp.float32),
            pltpu.VMEM((16,), jnp.int32),
        ],
        interpret=interpret,
    )
    def sc_kernel(p_hbm, eg_hbm, out_hbm, s0k, s0i, s1k, s1i,
                  cnt, va, vi, vb, sv, si, sp, t16, ti16):
        cid = lax.axis_index("c")
        sid = lax.axis_index("s")
        wid = sid * _NCORE + cid

        lane = lax.iota(jnp.int32, 16)
        lane0 = lane == 0
        zero16i = jnp.zeros((16,), jnp.int32)

        def zero_cnt():
            def body(i, _):
                cnt[pl.ds(i * 16, 16)] = zero16i
                return 0
            lax.fori_loop(0, _NBKT // 16, body, 0, unroll=4)

        def digit(kbits, shift):
            return 65535 - (lax.shift_right_logical(kbits, shift) & 65535)

        def dup_counts(d):
            key = d * 16 + lane
            ks, lanes_s = plsc.sort_key_val(key, lane)
            ds = lax.shift_right_logical(ks, 4)
            ti16[...] = ds
            prev = plsc.load_gather(ti16, [jnp.maximum(lane - 1, 0)])
            nxt = plsc.load_gather(ti16, [jnp.minimum(lane + 1, 15)])
            chg = (ds != prev) | (lane == 0)
            islast = (ds != nxt) | (lane == 15)
            runbase = plsc.cummax(jnp.where(chg, lane, 0))
            c = lane - runbase
            return ds, lanes_s, c, islast

        def hist_pass(src, rowbase, shift):
            def win(w, _):
                base = pl.multiple_of(rowbase + w * _W, 8)
                pltpu.sync_copy(src.at[pl.ds(base, _W)], va)

                def ch(t, _):
                    x = va[pl.ds(t * 16, 16)]
                    d = digit(plsc.bitcast(x, jnp.int32), shift)
                    ds, _, c, islast = dup_counts(d)
                    plsc.addupdate_scatter(cnt, [ds], c + 1, mask=islast)
                    return 0
                lax.fori_loop(0, _NCH, ch, 0)
                return 0
            lax.fori_loop(0, nwin, win, 0)

        def scan_cnt():
            def body(i, carry):
                x = cnt[pl.ds(i * 16, 16)]
                inc = plsc.cumsum(x)
                cnt[pl.ds(i * 16, 16)] = inc - x + carry
                return carry + jnp.max(inc)
            lax.fori_loop(0, _NBKT // 16, body, jnp.int32(0), unroll=4)

        def permute_pass(srcv, srci_or_none, dstv, dsti, rowbase, shift):
            def win(w, _):
                base = pl.multiple_of(rowbase + w * _W, 8)
                pltpu.sync_copy(srcv.at[pl.ds(base, _W)], va)
                if srci_or_none is not None:
                    pltpu.sync_copy(srci_or_none.at[pl.ds(base, _W)], vi)

                def ch(t, _):
                    x = va[pl.ds(t * 16, 16)]
                    d = digit(plsc.bitcast(x, jnp.int32), shift)
                    ds, lanes_s, c, islast = dup_counts(d)
                    off = plsc.load_gather(cnt, [ds])
                    plsc.addupdate_scatter(cnt, [ds], c + 1, mask=islast)
                    src_lane = t * 16 + lanes_s
                    sv[pl.ds(t * 16, 16)] = plsc.load_gather(va, [src_lane])
                    sp[pl.ds(t * 16, 16)] = off + c + rowbase
                    if srci_or_none is None:
                        si[pl.ds(t * 16, 16)] = w * _W + src_lane
                    else:
                        si[pl.ds(t * 16, 16)] = plsc.load_gather(
                            vi, [src_lane])
                    return 0
                lax.fori_loop(0, _NCH, ch, 0)
                pltpu.sync_copy(sv, dstv.at[sp])
                pltpu.sync_copy(si, dsti.at[sp])
                return 0
            lax.fori_loop(0, nwin, win, 0)

        def find_cutoff(rowbase):
            def win(w, st):
                base = pl.multiple_of(rowbase + w * _W, 8)
                pltpu.sync_copy(s1k.at[pl.ds(base, _W)], va)

                def ch(t, st):
                    carry, found, cutoff, z = st
                    x = va[pl.ds(t * 16, 16)]
                    inc = plsc.cumsum(x) + carry
                    crossed = inc >= _TOPP
                    anyc = jnp.max(crossed.astype(jnp.int32))
                    ffs = jnp.minimum(jnp.max(plsc.all_reduce_ffs(crossed)),
                                      jnp.int32(15))
                    t16[...] = inc
                    zv = jnp.max(plsc.load_gather(
                        t16, [jnp.broadcast_to(ffs, (16,))]))
                    take = (found == 0) & (anyc == 1)
                    cutoff = jnp.where(take, w * _W + t * 16 + ffs, cutoff)
                    z = jnp.where(take, zv, z)
                    found = found | anyc
                    return (jnp.max(inc), found, cutoff, z)
                return lax.fori_loop(0, _NCH, ch, st)
            st = (jnp.float32(0.0), jnp.int32(0), jnp.int32(_V - 1),
                  jnp.float32(1.0))
            _, _, cutoff, z = lax.fori_loop(0, nwin, win, st)
            return cutoff, z

        def sample_row(rowbase, cutoff, z):
            tinyz = jnp.float32(1e-30) * z

            def win(w, st):
                base = pl.multiple_of(rowbase + w * _W, 8)
                pltpu.sync_copy(s1k.at[pl.ds(base, _W)], va)
                pltpu.sync_copy(eg_hbm.at[pl.ds(base, _W)], vb)
                pltpu.sync_copy(s1i.at[pl.ds(base, _W)], vi)

                def ch(t, st):
                    bestv, bestj, bestidx = st
                    x = va[pl.ds(t * 16, 16)]
                    eg = vb[pl.ds(t * 16, 16)]
                    ix = vi[pl.ds(t * 16, 16)]
                    jvec = w * _W + t * 16 + lane
                    q = jnp.where(jvec <= cutoff, x,
                                  jnp.broadcast_to(tinyz, (16,)))
                    s = q * eg
                    upd = s > bestv
                    bestv = jnp.where(upd, s, bestv)
                    bestj = jnp.where(upd, jvec, bestj)
                    bestidx = jnp.where(upd, ix, bestidx)
                    return (bestv, bestj, bestidx)
                return lax.fori_loop(0, _NCH, ch, st)

            st = (jnp.full((16,), -1.0, jnp.float32), zero16i, zero16i)
            bestv, bestj, bestidx = lax.fori_loop(0, nwin, win, st)
            m = jnp.max(bestv)
            big = jnp.int32(2 ** 30)
            eqm = bestv == m
            jsel = jnp.min(jnp.where(eqm, bestj, big))
            wmask = eqm & (bestj == jsel)
            return jnp.min(jnp.where(wmask, bestidx, big))

        def write_row(rowbase, winner):
            winwin = winner // _W
            local = winner - winwin * _W
            fillv = jnp.full((16,), 1e-5, jnp.float32)
            bigv = jnp.full((16,), 1e5, jnp.float32)

            def initfill(t, _):
                va[pl.ds(t * 16, 16)] = fillv
                return 0
            lax.fori_loop(0, _NCH, initfill, 0, unroll=4)

            def win(w, _):
                base = pl.multiple_of(rowbase + w * _W, 8)
                inject = w == winwin
                idxv = jnp.broadcast_to(jnp.where(inject, local, 0), (16,))
                val = jnp.where(jnp.broadcast_to(inject, (16,)), bigv, fillv)
                plsc.store_scatter(va, [idxv], val, mask=lane0)
                pltpu.sync_copy(va, out_hbm.at[pl.ds(base, _W)])
                plsc.store_scatter(va, [idxv], fillv, mask=lane0)
                return 0
            lax.fori_loop(0, nwin, win, 0)

        def stage1(r, _):
            rowbase = (wid + _NWORK * r) * _V
            zero_cnt()
            hist_pass(p_hbm, rowbase, 0)
            scan_cnt()
            permute_pass(p_hbm, None, s0k, s0i, rowbase, 0)
            return 0

        def stage2(r, _):
            rowbase = (wid + _NWORK * r) * _V
            zero_cnt()
            hist_pass(s0k, rowbase, 16)
            scan_cnt()
            permute_pass(s0k, s0i, s1k, s1i, rowbase, 16)
            return 0

        def stage3(r, _):
            rowbase = (wid + _NWORK * r) * _V
            cutoff, z = find_cutoff(rowbase)
            winner = sample_row(rowbase, cutoff, z)
            write_row(rowbase, winner)
            return 0

        nrows = _B // _NWORK
        lax.fori_loop(0, nrows, stage1, 0)
        plsc.subcore_barrier()
        lax.fori_loop(0, nrows, stage2, 0)
        plsc.subcore_barrier()
        lax.fori_loop(0, nrows, stage3, 0)

    return sc_kernel


def kernel(input_ids, logits, g_values):
    del input_ids
    probs = jax.nn.softmax(logits, axis=-1)
    g = g_values.astype(jnp.float32)
    for i in range(_DEPTH):
        g_d = g[:, i, :]
        g_mass = jnp.sum(g_d * probs, axis=-1, keepdims=True)
        probs = probs * (1.0 + g_d - g_mass)
    eg = _expg_const()
    sc = _build_sc()
    out_flat, _, _, _, _ = sc(probs.reshape(-1), eg.reshape(-1))
    return out_flat.reshape(_B, _V)


# speedup vs baseline: 1.0078x; 1.0078x over previous
import functools

import jax
import jax.numpy as jnp
from jax import lax
from jax.experimental import pallas as pl
from jax.experimental.pallas import tpu as pltpu
from jax.experimental.pallas import tpu_sc as plsc

_B = 64
_V = 100000
_DEPTH = 4
_TOPP = 0.9

_W = 10000
_NCH = _W // 16
_NBKT = 65536
_NCORE = 2
_NSUB = 16
_NWORK = _NCORE * _NSUB


def _expg_const():
    skey = jax.random.key(12345)
    skeys = jax.random.split(skey, _B)
    g = jax.vmap(lambda k: jax.random.gumbel(k, (_V,), jnp.float32))(skeys)
    return jnp.exp(g)


def _build_sc(interpret=False):
    nwin = _V // _W
    mesh = plsc.VectorSubcoreMesh(
        core_axis_name="c", subcore_axis_name="s",
        num_cores=_NCORE, num_subcores=_NSUB)

    @functools.partial(
        pl.kernel,
        mesh=mesh,
        compiler_params=pltpu.CompilerParams(needs_layout_passes=False),
        out_type=[
            jax.ShapeDtypeStruct((_B * _V,), jnp.float32),
            jax.ShapeDtypeStruct((_B * _V,), jnp.float32),
            jax.ShapeDtypeStruct((_B * _V,), jnp.int32),
            jax.ShapeDtypeStruct((_B * _V,), jnp.float32),
            jax.ShapeDtypeStruct((_B * _V,), jnp.int32),
        ],
        scratch_types=[
            pltpu.VMEM((_NBKT,), jnp.int32),
            pltpu.VMEM((_W,), jnp.float32),
            pltpu.VMEM((_W,), jnp.int32),
            pltpu.VMEM((_W,), jnp.float32),
            pltpu.VMEM((_W,), jnp.float32),
            pltpu.VMEM((_W,), jnp.int32),
            pltpu.VMEM((_W,), jnp.int32),
            pltpu.VMEM((16,), jnp.float32),
            pltpu.VMEM((16,), jnp.int32),
        ],
        interpret=interpret,
    )
    def sc_kernel(p_hbm, eg_hbm, out_hbm, s0k, s0i, s1k, s1i,
                  cnt, va, vi, vb, sv, si, sp, t16, ti16):
        cid = lax.axis_index("c")
        sid = lax.axis_index("s")
        wid = sid * _NCORE + cid

        lane = lax.iota(jnp.int32, 16)
        lane0 = lane == 0
        zero16i = jnp.zeros((16,), jnp.int32)

        def zero_cnt():
            def body(i, _):
                cnt[pl.ds(i * 16, 16)] = zero16i
                return 0
            lax.fori_loop(0, _NBKT // 16, body, 0, unroll=4)

        def digit(kbits, shift):
            return 65535 - (lax.shift_right_logical(kbits, shift) & 65535)

        def dup_counts(d):
            key = d * 16 + lane
            ks, lanes_s = plsc.sort_key_val(key, lane)
            ds = lax.shift_right_logical(ks, 4)
            ti16[...] = ds
            prev = plsc.load_gather(ti16, [jnp.maximum(lane - 1, 0)])
            nxt = plsc.load_gather(ti16, [jnp.minimum(lane + 1, 15)])
            chg = (ds != prev) | (lane == 0)
            islast = (ds != nxt) | (lane == 15)
            runbase = plsc.cummax(jnp.where(chg, lane, 0))
            c = lane - runbase
            return ds, lanes_s, c, islast

        def hist_pass(src, rowbase, shift):
            def win(w, _):
                base = pl.multiple_of(rowbase + w * _W, 8)
                pltpu.sync_copy(src.at[pl.ds(base, _W)], va)

                def ch(t, _):
                    x = va[pl.ds(t * 16, 16)]
                    d = digit(plsc.bitcast(x, jnp.int32), shift)
                    ds, _, c, islast = dup_counts(d)
                    plsc.addupdate_scatter(cnt, [ds], c + 1, mask=islast)
                    return 0
                lax.fori_loop(0, _NCH, ch, 0)
                return 0
            lax.fori_loop(0, nwin, win, 0)

        def scan_cnt():
            def body(i, carry):
                x = cnt[pl.ds(i * 16, 16)]
                inc = plsc.cumsum(x)
                cnt[pl.ds(i * 16, 16)] = inc - x + carry
                return carry + jnp.max(inc)
            lax.fori_loop(0, _NBKT // 16, body, jnp.int32(0), unroll=4)

        def permute_pass(srcv, srci_or_none, dstv, dsti, rowbase, shift):
            def win(w, _):
                base = pl.multiple_of(rowbase + w * _W, 8)
                pltpu.sync_copy(srcv.at[pl.ds(base, _W)], va)
                if srci_or_none is not None:
                    pltpu.sync_copy(srci_or_none.at[pl.ds(base, _W)], vi)

                def ch(t, _):
                    x = va[pl.ds(t * 16, 16)]
                    d = digit(plsc.bitcast(x, jnp.int32), shift)
                    ds, lanes_s, c, islast = dup_counts(d)
                    off = plsc.load_gather(cnt, [ds])
                    plsc.addupdate_scatter(cnt, [ds], c + 1, mask=islast)
                    src_lane = t * 16 + lanes_s
                    sv[pl.ds(t * 16, 16)] = plsc.load_gather(va, [src_lane])
                    sp[pl.ds(t * 16, 16)] = off + c + rowbase
                    if srci_or_none is None:
                        si[pl.ds(t * 16, 16)] = w * _W + src_lane
                    else:
                        si[pl.ds(t * 16, 16)] = plsc.load_gather(
                            vi, [src_lane])
                    return 0
                lax.fori_loop(0, _NCH, ch, 0)
                pltpu.sync_copy(sv, dstv.at[sp])
                pltpu.sync_copy(si, dsti.at[sp])
                return 0
            lax.fori_loop(0, nwin, win, 0)

        def find_cutoff(rowbase):
            def win(w, st):
                base = pl.multiple_of(rowbase + w * _W, 8)
                pltpu.sync_copy(s1k.at[pl.ds(base, _W)], va)

                def ch(t, st):
                    carry, found, cutoff, z = st
                    x = va[pl.ds(t * 16, 16)]
                    inc = plsc.cumsum(x) + carry
                    crossed = inc >= _TOPP
                    anyc = jnp.max(crossed.astype(jnp.int32))
                    ffs = jnp.minimum(jnp.max(plsc.all_reduce_ffs(crossed)),
                                      jnp.int32(15))
                    t16[...] = inc
                    zv = jnp.max(plsc.load_gather(
                        t16, [jnp.broadcast_to(ffs, (16,))]))
                    take = (found == 0) & (anyc == 1)
                    cutoff = jnp.where(take, w * _W + t * 16 + ffs, cutoff)
                    z = jnp.where(take, zv, z)
                    found = found | anyc
                    return (jnp.max(inc), found, cutoff, z)
                return lax.fori_loop(0, _NCH, ch, st)
            st = (jnp.float32(0.0), jnp.int32(0), jnp.int32(_V - 1),
                  jnp.float32(1.0))
            _, _, cutoff, z = lax.fori_loop(0, nwin, win, st)
            return cutoff, z

        def sample_row(rowbase, cutoff, z):
            tinyz = jnp.float32(1e-30) * z

            def win(w, st):
                base = pl.multiple_of(rowbase + w * _W, 8)
                pltpu.sync_copy(s1k.at[pl.ds(base, _W)], va)
                pltpu.sync_copy(eg_hbm.at[pl.ds(base, _W)], vb)
                pltpu.sync_copy(s1i.at[pl.ds(base, _W)], vi)

                def ch(t, st):
                    bestv, bestj, bestidx = st
                    x = va[pl.ds(t * 16, 16)]
                    eg = vb[pl.ds(t * 16, 16)]
                    ix = vi[pl.ds(t * 16, 16)]
                    jvec = w * _W + t * 16 + lane
                    q = jnp.where(jvec <= cutoff, x,
                                  jnp.broadcast_to(tinyz, (16,)))
                    s = q * eg
                    upd = s > bestv
                    bestv = jnp.where(upd, s, bestv)
                    bestj = jnp.where(upd, jvec, bestj)
                    bestidx = jnp.where(upd, ix, bestidx)
                    return (bestv, bestj, bestidx)
                return lax.fori_loop(0, _NCH, ch, st)

            st = (jnp.full((16,), -1.0, jnp.float32), zero16i, zero16i)
            bestv, bestj, bestidx = lax.fori_loop(0, nwin, win, st)
            m = jnp.max(bestv)
            big = jnp.int32(2 ** 30)
            eqm = bestv == m
            jsel = jnp.min(jnp.where(eqm, bestj, big))
            wmask = eqm & (bestj == jsel)
            return jnp.min(jnp.where(wmask, bestidx, big))

        def write_row(rowbase, winner):
            winwin = winner // _W
            local = winner - winwin * _W
            fillv = jnp.full((16,), 1e-5, jnp.float32)
            bigv = jnp.full((16,), 1e5, jnp.float32)

            def initfill(t, _):
                va[pl.ds(t * 16, 16)] = fillv
                return 0
            lax.fori_loop(0, _NCH, initfill, 0, unroll=4)

            def win(w, _):
                base = pl.multiple_of(rowbase + w * _W, 8)
                inject = w == winwin
                idxv = jnp.broadcast_to(jnp.where(inject, local, 0), (16,))
                val = jnp.where(jnp.broadcast_to(inject, (16,)), bigv, fillv)
                plsc.store_scatter(va, [idxv], val, mask=lane0)
                pltpu.sync_copy(va, out_hbm.at[pl.ds(base, _W)])
                plsc.store_scatter(va, [idxv], fillv, mask=lane0)
                return 0
            lax.fori_loop(0, nwin, win, 0)

        def stage1(r, _):
            rowbase = (wid + _NWORK * r) * _V
            zero_cnt()
            hist_pass(p_hbm, rowbase, 0)
            scan_cnt()
            permute_pass(p_hbm, None, s0k, s0i, rowbase, 0)
            return 0

        def stage2(r, _):
            rowbase = (wid + _NWORK * r) * _V
            zero_cnt()
            hist_pass(s0k, rowbase, 16)
            scan_cnt()
            permute_pass(s0k, s0i, s1k, s1i, rowbase, 16)
            return 0

        def stage3(r, _):
            rowbase = (wid + _NWORK * r) * _V
            cutoff, z = find_cutoff(rowbase)
            winner = sample_row(rowbase, cutoff, z)
            write_row(rowbase, winner)
            return 0

        nrows = _B // _NWORK
        lax.fori_loop(0, nrows, stage1, 0)
        plsc.subcore_barrier()
        lax.fori_loop(0, nrows, stage2, 0)
        plsc.subcore_barrier()
        lax.fori_loop(0, nrows, stage3, 0)

    return sc_kernel


def kernel(input_ids, logits, g_values):
    del input_ids
    probs = jax.nn.softmax(logits, axis=-1)
    g = g_values.astype(jnp.float32)
    for i in range(_DEPTH):
        g_d = g[:, i, :]
        g_mass = jnp.sum(g_d * probs, axis=-1, keepdims=True)
        probs = probs * (1.0 + g_d - g_mass)
    eg = _expg_const()
    sc = _build_sc()
    out_flat, _, _, _, _ = sc(probs.reshape(-1), eg.reshape(-1))
    return out_flat.reshape(_B, _V)
